# Initial kernel scaffold; baseline (speedup 1.0000x reference)
import jax
import jax.numpy as jnp
from jax.experimental import pallas as pl


def kernel(cuj, user_id, target, neg_p, History, distance, ng_distance, a, b, UserPreference, PoiPreference, GeoInfluence, GeoSusceptibility):
    raise NotImplementedError("write your pallas kernel here")


# speedup vs baseline: 4.5159x; 4.5159x over previous
import functools

import jax
import jax.numpy as jnp
from jax import lax
from jax.experimental import pallas as pl
from jax.experimental.pallas import tpu as pltpu
from jax.experimental.pallas import tpu_sc as plsc

EMB = 16
NEG = 20
HIST = 200
NROW = 24
BLK = 128
NC, NS = 2, 16
NW = NC * NS
GSLOT = 256
P_OFF, H_OFF, U_OFF = GSLOT, GSLOT + 32, GSLOT + 64
NROWS_PER_W = 11


def _sc_gather_body(gi_hbm, pp_hbm, gs_hbm, up_hbm, idx_hbm, out_hbm,
                    gidx_v, p1_v, p2_v, h1_v, h2_v, u_v,
                    blk_v, row_v, isem, bsem):
    wid = lax.axis_index("s") * NC + lax.axis_index("c")
    base = wid * 8
    lanes16 = lax.iota(jnp.int32, 16)

    iloads = [
        pltpu.async_copy(idx_hbm.at[pl.ds(base, 16)], gidx_v, isem),
        pltpu.async_copy(idx_hbm.at[pl.ds(P_OFF, 16)], p1_v, isem),
        pltpu.async_copy(idx_hbm.at[pl.ds(P_OFF + 16, 16)], p2_v, isem),
        pltpu.async_copy(idx_hbm.at[pl.ds(H_OFF, 16)], h1_v, isem),
        pltpu.async_copy(idx_hbm.at[pl.ds(H_OFF + 16, 16)], h2_v, isem),
        pltpu.async_copy(idx_hbm.at[pl.ds(U_OFF, 16)], u_v, isem),
    ]
    for c in iloads:
        c.wait()

    def extract(vec, k):
        return jnp.sum(jnp.where(lanes16 == k, vec, 0))

    hi = wid >= 16
    sel = wid & 15
    gvec = gidx_v[...]
    pvec = jnp.where(hi, p2_v[...], p1_v[...])
    hvec = jnp.where(hi, h2_v[...], h1_v[...])
    rs = [extract(gvec, k) for k in range(8)]
    rs.append(extract(pvec, sel))
    rs.append(extract(hvec, sel))
    rs.append(extract(u_v[...], 0))
    tbls = [gi_hbm] * 8 + [pp_hbm, gs_hbm, up_hbm]

    copies = []
    for k in range(NROWS_PER_W):
        col0 = pl.multiple_of(rs[k] & ~(BLK - 1), BLK)
        copies.append(pltpu.async_copy(
            tbls[k].at[:, pl.ds(col0, BLK)], blk_v.at[k], bsem))
    for c in copies:
        c.wait()
    for k in range(NROWS_PER_W):
        lane_vec = jnp.full((16,), rs[k] & (BLK - 1), jnp.int32)
        row_v[0, k] = plsc.load_gather(blk_v.at[k], [lanes16, lane_vec])

    pltpu.sync_copy(row_v, out_hbm.at[pl.ds(wid, 1)])


@functools.cache
def _sc_gather_kernel():
    return pl.kernel(
        _sc_gather_body,
        mesh=plsc.VectorSubcoreMesh(core_axis_name="c", subcore_axis_name="s"),
        out_type=jax.ShapeDtypeStruct((NW, NROWS_PER_W, EMB), jnp.float32),
        scratch_types=[
            pltpu.VMEM((16,), jnp.int32),
            pltpu.VMEM((16,), jnp.int32),
            pltpu.VMEM((16,), jnp.int32),
            pltpu.VMEM((16,), jnp.int32),
            pltpu.VMEM((16,), jnp.int32),
            pltpu.VMEM((16,), jnp.int32),
            pltpu.VMEM((NROWS_PER_W, EMB, BLK), jnp.float32),
            pltpu.VMEM((1, NROWS_PER_W, EMB), jnp.float32),
            pltpu.SemaphoreType.DMA,
            pltpu.SemaphoreType.DMA,
        ],
        compiler_params=pltpu.CompilerParams(disable_bounds_checks=True,
                                             needs_layout_passes=False),
    )


def _tc_body(scal_ref, dall_ref, rows_ref, out_ref):
    a = scal_ref[0, 0]
    b = scal_ref[0, 1]
    cujf = scal_ref[0, 2]
    g = jnp.reshape(rows_ref[:, 0:8, :], (GSLOT, EMB))[0:HIST]
    p = jnp.reshape(rows_ref[0:NROW, 8:9, :], (NROW, EMB))
    h = jnp.reshape(rows_ref[0:NROW, 9:10, :], (NROW, EMB))
    u8 = jnp.reshape(rows_ref[0:8, 10:11, :], (8, EMB))
    ulane = lax.broadcasted_iota(jnp.int32, (8, EMB), 0)
    u_m = jnp.where(ulane == 0, u8, 0.0)
    d = dall_ref[...]
    f = a * jnp.power(d, b)
    s = lax.dot_general(h, g, (((1,), (1,)), ((), ())),
                        preferred_element_type=jnp.float32)
    y = jnp.sum(f * s, axis=1, keepdims=True) * (1.0 / HIST)
    tz = lax.dot_general(p, u_m, (((1,), (1,)), ((), ())),
                         preferred_element_type=jnp.float32)
    t = jnp.sum(tz, axis=1, keepdims=True) + y
    rowid = lax.broadcasted_iota(jnp.int32, (NROW, 1), 0)
    x = jnp.where(rowid == 0, -t, t)
    sp = jnp.maximum(x, 0.0) + jnp.log1p(jnp.exp(-jnp.abs(x)))
    valid = (rowid < NEG + 1).astype(jnp.float32)
    loss = jnp.sum(-sp * valid)
    wuj = 1.0 + jnp.log(1.0 + cujf * (10.0 ** 10))
    out_ref[...] = jnp.full((1, 1), -wuj * loss, jnp.float32)


_tc_call = pl.pallas_call(
    _tc_body,
    out_shape=jax.ShapeDtypeStruct((1, 1), jnp.float32),
    in_specs=[
        pl.BlockSpec(memory_space=pltpu.SMEM),
        pl.BlockSpec(memory_space=pltpu.VMEM),
        pl.BlockSpec(memory_space=pltpu.VMEM),
    ],
    out_specs=pl.BlockSpec(memory_space=pltpu.VMEM),
)


def kernel(cuj, user_id, target, neg_p, History, distance, ng_distance,
           a, b, UserPreference, PoiPreference, GeoInfluence,
           GeoSusceptibility):
    uid = jnp.asarray(user_id, jnp.int32)
    tn = jnp.concatenate([jnp.asarray(target, jnp.int32).reshape(1),
                          jnp.asarray(neg_p, jnp.int32).reshape(NEG),
                          jnp.zeros((32 - NEG - 1,), jnp.int32)])
    idx_all = jnp.concatenate([
        History.astype(jnp.int32).reshape(HIST),
        jnp.zeros((GSLOT - HIST,), jnp.int32),
        tn,
        tn,
        jnp.full((16,), uid, jnp.int32),
    ])

    rows = _sc_gather_kernel()(
        GeoInfluence.T, PoiPreference.T, GeoSusceptibility.T,
        UserPreference.T, idx_all)

    dall = jnp.concatenate([
        distance.astype(jnp.float32).reshape(1, HIST),
        ng_distance.astype(jnp.float32).reshape(NEG, HIST),
        jnp.ones((NROW - NEG - 1, HIST), jnp.float32),
    ])
    scal = jnp.stack([a[0].astype(jnp.float32), b[0].astype(jnp.float32),
                      jnp.asarray(cuj, jnp.float32)]).reshape(1, 3)

    return _tc_call(scal, dall, rows)


# speedup vs baseline: 4.5892x; 1.0162x over previous
import functools

import jax
import jax.numpy as jnp
from jax import lax
from jax.experimental import pallas as pl
from jax.experimental.pallas import tpu as pltpu
from jax.experimental.pallas import tpu_sc as plsc

EMB = 16
NEG = 20
HIST = 200
POI = 1000000
BLK = 128
NC, NS = 2, 16
NW = NC * NS
GSLOT = 256
NROWS_PER_W = 11


def _sc_gather_body(gi_hbm, pp_hbm, gs_hbm, up_hbm, hist_hbm, sidx_hbm,
                    out_hbm,
                    gidx_v, s1_v, s2_v, blk_v, row_v, isem, bsem):
    wid = lax.axis_index("s") * NC + lax.axis_index("c")
    gbase = jnp.minimum(wid * 8, 192)
    lanes16 = lax.iota(jnp.int32, 16)

    iloads = [
        pltpu.async_copy(hist_hbm.at[pl.ds(gbase, 16)], gidx_v, isem),
        pltpu.async_copy(sidx_hbm.at[pl.ds(0, 16)], s1_v, isem),
        pltpu.async_copy(sidx_hbm.at[pl.ds(16, 16)], s2_v, isem),
    ]
    for c in iloads:
        c.wait()

    def extract(vec, k):
        return jnp.sum(jnp.where(lanes16 == k, vec, 0))

    svec = jnp.where(wid >= 16, s2_v[...], s1_v[...])
    rp = extract(svec, wid & 15)

    gvec = gidx_v[...]
    rs = [extract(gvec, k) for k in range(8)]
    rs.append(rp)
    rs.append(rp)
    rs.append(extract(s2_v[...], 5))
    rs = [lax.clamp(0, r, POI - 1) for r in rs]
    tbls = [gi_hbm] * 8 + [pp_hbm, gs_hbm, up_hbm]

    copies = []
    for k in range(NROWS_PER_W):
        col0 = pl.multiple_of(rs[k] & ~(BLK - 1), BLK)
        copies.append(pltpu.async_copy(
            tbls[k].at[:, pl.ds(col0, BLK)], blk_v.at[k], bsem))
    for c in copies:
        c.wait()
    for k in range(NROWS_PER_W):
        lane_vec = jnp.full((16,), rs[k] & (BLK - 1), jnp.int32)
        row_v[0, k] = plsc.load_gather(blk_v.at[k], [lanes16, lane_vec])

    pltpu.sync_copy(row_v, out_hbm.at[pl.ds(wid, 1)])


@functools.cache
def _sc_gather_kernel():
    return pl.kernel(
        _sc_gather_body,
        mesh=plsc.VectorSubcoreMesh(core_axis_name="c", subcore_axis_name="s"),
        out_type=jax.ShapeDtypeStruct((NW, NROWS_PER_W, EMB), jnp.float32),
        scratch_types=[
            pltpu.VMEM((16,), jnp.int32),
            pltpu.VMEM((16,), jnp.int32),
            pltpu.VMEM((16,), jnp.int32),
            pltpu.VMEM((NROWS_PER_W, EMB, BLK), jnp.float32),
            pltpu.VMEM((1, NROWS_PER_W, EMB), jnp.float32),
            pltpu.SemaphoreType.DMA,
            pltpu.SemaphoreType.DMA,
        ],
        compiler_params=pltpu.CompilerParams(disable_bounds_checks=True,
                                             needs_layout_passes=False),
    )


def _tc_body(a_ref, b_ref, cuj_ref, d_ref, ngd_ref, rows_ref, out_ref):
    a = a_ref[0]
    b = b_ref[0]
    cujf = cuj_ref[0].astype(jnp.float32)
    g = jnp.reshape(rows_ref[:, 0:8, :], (GSLOT, EMB))[0:HIST]
    p = jnp.reshape(rows_ref[:, 8:9, :], (NW, EMB))
    h = jnp.reshape(rows_ref[:, 9:10, :], (NW, EMB))
    u8 = jnp.reshape(rows_ref[0:8, 10:11, :], (8, EMB))
    ulane = lax.broadcasted_iota(jnp.int32, (8, EMB), 0)
    u_m = jnp.where(ulane == 0, u8, 0.0)
    fd = a * jnp.power(d_ref[...], b)
    fng = a * jnp.power(ngd_ref[...], b)
    s = lax.dot_general(h, g, (((1,), (1,)), ((), ())),
                        preferred_element_type=jnp.float32)
    tz = lax.dot_general(p, u_m, (((1,), (1,)), ((), ())),
                         preferred_element_type=jnp.float32)
    ts = jnp.sum(tz, axis=1, keepdims=True)
    y0 = jnp.sum(fd * s[0:1], axis=1, keepdims=True) * (1.0 / HIST)
    yng = jnp.sum(fng * s[1:NEG + 1], axis=1, keepdims=True) * (1.0 / HIST)
    t0 = -(ts[0:1] + y0)
    tng = ts[1:NEG + 1] + yng
    sp0 = jnp.maximum(t0, 0.0) + jnp.log1p(jnp.exp(-jnp.abs(t0)))
    spn = jnp.maximum(tng, 0.0) + jnp.log1p(jnp.exp(-jnp.abs(tng)))
    loss = -(jnp.sum(sp0) + jnp.sum(spn))
    wuj = 1.0 + jnp.log(1.0 + cujf * (10.0 ** 10))
    out_ref[...] = jnp.full((1, 1), -wuj * loss, jnp.float32)


_tc_call = pl.pallas_call(
    _tc_body,
    out_shape=jax.ShapeDtypeStruct((1, 1), jnp.float32),
    in_specs=[
        pl.BlockSpec(memory_space=pltpu.SMEM),
        pl.BlockSpec(memory_space=pltpu.SMEM),
        pl.BlockSpec(memory_space=pltpu.SMEM),
        pl.BlockSpec(memory_space=pltpu.VMEM),
        pl.BlockSpec(memory_space=pltpu.VMEM),
        pl.BlockSpec(memory_space=pltpu.VMEM),
    ],
    out_specs=pl.BlockSpec(memory_space=pltpu.VMEM),
)


def kernel(cuj, user_id, target, neg_p, History, distance, ng_distance,
           a, b, UserPreference, PoiPreference, GeoInfluence,
           GeoSusceptibility):
    sidx = jnp.concatenate([
        jnp.asarray(target, jnp.int32).reshape(1),
        jnp.asarray(neg_p, jnp.int32).reshape(NEG),
        jnp.full((11,), user_id, jnp.int32),
    ])
    rows = _sc_gather_kernel()(
        GeoInfluence.T, PoiPreference.T, GeoSusceptibility.T,
        UserPreference.T, History.astype(jnp.int32), sidx)

    cuj_a = jnp.reshape(jnp.asarray(cuj, jnp.int32), (1,))
    return _tc_call(a, b, cuj_a, distance.reshape(1, HIST), ng_distance,
                    rows)


# speedup vs baseline: 5.0905x; 1.1092x over previous
import functools

import jax
import jax.numpy as jnp
from jax import lax
from jax.experimental import pallas as pl
from jax.experimental.pallas import tpu as pltpu
from jax.experimental.pallas import tpu_sc as plsc

EMB = 16
NEG = 20
HIST = 200
POI = 1000000
BLK = 128
NC, NS = 2, 16
NW = NC * NS
GSLOT = 256
NROWS_PER_W = 11


def _sc_gather_body(gi_hbm, pp_hbm, gs_hbm, up_hbm, hist_hbm, sidx_hbm,
                    out_hbm,
                    gidx_v, s1_v, s2_v, blk_v, row_v, isem, bsem):
    wid = lax.axis_index("s") * NC + lax.axis_index("c")
    gbase = jnp.minimum(wid * 8, 192)
    lanes16 = lax.iota(jnp.int32, 16)

    iloads = [
        pltpu.async_copy(hist_hbm.at[pl.ds(gbase, 16)], gidx_v, isem),
        pltpu.async_copy(sidx_hbm.at[pl.ds(0, 16)], s1_v, isem),
        pltpu.async_copy(sidx_hbm.at[pl.ds(16, 16)], s2_v, isem),
    ]
    for c in iloads:
        c.wait()

    def extract(vec, k):
        return jnp.sum(jnp.where(lanes16 == k, vec, 0))

    svec = jnp.where(wid >= 16, s2_v[...], s1_v[...])
    rp = extract(svec, wid & 15)

    gvec = gidx_v[...]
    rs = [extract(gvec, k) for k in range(8)]
    rs.append(rp)
    rs.append(rp)
    rs.append(extract(s2_v[...], 5))
    rs = [lax.clamp(0, r, POI - 1) for r in rs]
    tbls = [gi_hbm] * 8 + [pp_hbm, gs_hbm, up_hbm]

    def fire(k):
        col0 = pl.multiple_of(rs[k] & ~(BLK - 1), BLK)
        return pltpu.async_copy(
            tbls[k].at[:, pl.ds(col0, BLK)], blk_v.at[k], bsem)

    @pl.when(wid < 25)
    def _():
        gh = [fire(k) for k in range(8)]

        @pl.when(wid < NEG + 1)
        def _():
            ph = [fire(8), fire(9)]

            @pl.when(wid == 0)
            def _():
                fire(10).wait()

            for c in ph:
                c.wait()

        for c in gh:
            c.wait()
    for k in range(NROWS_PER_W):
        lane_vec = jnp.full((16,), rs[k] & (BLK - 1), jnp.int32)
        row_v[0, k] = plsc.load_gather(blk_v.at[k], [lanes16, lane_vec])

    pltpu.sync_copy(row_v, out_hbm.at[pl.ds(wid, 1)])


@functools.cache
def _sc_gather_kernel():
    return pl.kernel(
        _sc_gather_body,
        mesh=plsc.VectorSubcoreMesh(core_axis_name="c", subcore_axis_name="s"),
        out_type=jax.ShapeDtypeStruct((NW, NROWS_PER_W, EMB), jnp.float32),
        scratch_types=[
            pltpu.VMEM((16,), jnp.int32),
            pltpu.VMEM((16,), jnp.int32),
            pltpu.VMEM((16,), jnp.int32),
            pltpu.VMEM((NROWS_PER_W, EMB, BLK), jnp.float32),
            pltpu.VMEM((1, NROWS_PER_W, EMB), jnp.float32),
            pltpu.SemaphoreType.DMA,
            pltpu.SemaphoreType.DMA,
        ],
        compiler_params=pltpu.CompilerParams(disable_bounds_checks=True,
                                             needs_layout_passes=False),
    )


def _tc_body(a_ref, b_ref, cuj_ref, d_ref, ngd_ref, rows_ref, out_ref):
    a = a_ref[0]
    b = b_ref[0]
    cujf = cuj_ref[0].astype(jnp.float32)
    g = jnp.reshape(rows_ref[:, 0:8, :], (GSLOT, EMB))[0:HIST]
    p = jnp.reshape(rows_ref[:, 8:9, :], (NW, EMB))
    h = jnp.reshape(rows_ref[:, 9:10, :], (NW, EMB))
    u8 = jnp.reshape(rows_ref[0:8, 10:11, :], (8, EMB))
    ulane = lax.broadcasted_iota(jnp.int32, (8, EMB), 0)
    u_m = jnp.where(ulane == 0, u8, 0.0)
    fd = a * jnp.power(d_ref[...], b)
    fng = a * jnp.power(ngd_ref[...], b)
    s = lax.dot_general(h, g, (((1,), (1,)), ((), ())),
                        preferred_element_type=jnp.float32)
    tz = lax.dot_general(p, u_m, (((1,), (1,)), ((), ())),
                         preferred_element_type=jnp.float32)
    ts = jnp.sum(tz, axis=1, keepdims=True)
    y0 = jnp.sum(fd * s[0:1], axis=1, keepdims=True) * (1.0 / HIST)
    yng = jnp.sum(fng * s[1:NEG + 1], axis=1, keepdims=True) * (1.0 / HIST)
    t0 = -(ts[0:1] + y0)
    tng = ts[1:NEG + 1] + yng
    sp0 = jnp.maximum(t0, 0.0) + jnp.log1p(jnp.exp(-jnp.abs(t0)))
    spn = jnp.maximum(tng, 0.0) + jnp.log1p(jnp.exp(-jnp.abs(tng)))
    loss = -(jnp.sum(sp0) + jnp.sum(spn))
    wuj = 1.0 + jnp.log(1.0 + cujf * (10.0 ** 10))
    out_ref[...] = jnp.full((1, 1), -wuj * loss, jnp.float32)


_tc_call = pl.pallas_call(
    _tc_body,
    out_shape=jax.ShapeDtypeStruct((1, 1), jnp.float32),
    in_specs=[
        pl.BlockSpec(memory_space=pltpu.SMEM),
        pl.BlockSpec(memory_space=pltpu.SMEM),
        pl.BlockSpec(memory_space=pltpu.SMEM),
        pl.BlockSpec(memory_space=pltpu.VMEM),
        pl.BlockSpec(memory_space=pltpu.VMEM),
        pl.BlockSpec(memory_space=pltpu.VMEM),
    ],
    out_specs=pl.BlockSpec(memory_space=pltpu.VMEM),
)


def kernel(cuj, user_id, target, neg_p, History, distance, ng_distance,
           a, b, UserPreference, PoiPreference, GeoInfluence,
           GeoSusceptibility):
    sidx = jnp.concatenate([
        jnp.asarray(target, jnp.int32).reshape(1),
        jnp.asarray(neg_p, jnp.int32).reshape(NEG),
        jnp.full((11,), user_id, jnp.int32),
    ])
    rows = _sc_gather_kernel()(
        GeoInfluence.T, PoiPreference.T, GeoSusceptibility.T,
        UserPreference.T, History.astype(jnp.int32), sidx)

    cuj_a = jnp.reshape(jnp.asarray(cuj, jnp.int32), (1,))
    return _tc_call(a, b, cuj_a, distance.reshape(1, HIST), ng_distance,
                    rows)
